# Initial kernel scaffold; baseline (speedup 1.0000x reference)
import functools

import jax
import jax.numpy as jnp
from jax import lax
from jax.experimental import pallas as pl
from jax.experimental.pallas import tpu as pltpu
from jax.experimental.pallas import tpu_sc as plsc

N_NODES = 10000
D = 128
E = 320000

NC = 2
NS = 16
NW = NC * NS
EPW = E // NW
CH = 80
NCH = EPW // CH
ROWS_PER_TILE = N_NODES // NS
ZR = 125

_mesh = plsc.VectorSubcoreMesh(core_axis_name="c", subcore_axis_name="s")


@functools.partial(
    pl.kernel,
    mesh=_mesh,
    out_type=(
        jax.ShapeDtypeStruct((NC, N_NODES, D), jnp.float32),
        jax.ShapeDtypeStruct((NW, N_NODES), jnp.float32),
    ),
    scratch_types=[
        pltpu.VMEM((NCH, CH), jnp.int32),
        pltpu.VMEM((NCH, CH), jnp.int32),
        pltpu.VMEM((CH, D), jnp.float32),
        pltpu.VMEM((CH, D), jnp.float32),
        pltpu.VMEM((ZR, D), jnp.float32),
        pltpu.VMEM((N_NODES,), jnp.float32),
        pltpu.VMEM_SHARED((N_NODES, D), jnp.float32),
        pltpu.SemaphoreType.DMA,
        pltpu.SemaphoreType.DMA,
    ],
)
def _sc_agg(x_hbm, src_hbm, dst_hbm, agg_out, cnt_out,
            src_v, dst_v, rows_a, rows_b, zbuf, cnt_v, acc, sem_a, sem_b):
    cid = lax.axis_index("c")
    tid = lax.axis_index("s")
    wid = cid * NS + tid

    pltpu.sync_copy(src_hbm.at[wid], src_v)
    pltpu.sync_copy(dst_hbm.at[wid], dst_v)

    zvec = jnp.zeros((16,), jnp.float32)

    def zero_cnt(i, carry):
        cnt_v[pl.ds(i * 16, 16)] = zvec
        return carry

    lax.fori_loop(0, N_NODES // 16, zero_cnt, 0)

    def zero_zbuf(r, carry):
        row = zbuf.at[r]
        for k in range(D // 16):
            row[pl.ds(k * 16, 16)] = zvec
        return carry

    lax.fori_loop(0, ZR, zero_zbuf, 0)

    base = tid * ROWS_PER_TILE

    def zero_acc(b, carry):
        pltpu.sync_copy(zbuf, acc.at[pl.ds(base + b * ZR, ZR)])
        return carry

    lax.fori_loop(0, ROWS_PER_TILE // ZR, zero_acc, 0)

    plsc.subcore_barrier()

    ones16 = jnp.ones((16,), jnp.float32)

    def gather(c, buf, sem):
        return pltpu.make_async_copy(x_hbm.at[src_v.at[c]], buf, sem)

    def do_chunk(c, buf):
        pltpu.sync_copy(buf, acc.at[dst_v.at[c]], add=True)
        drow = dst_v.at[c]
        for k in range(CH // 16):
            dk = drow[pl.ds(k * 16, 16)]
            plsc.addupdate_scatter(cnt_v, [dk], ones16)

    gather(0, rows_a, sem_a).start()

    def step(t, carry):
        c0 = 2 * t
        gather(c0 + 1, rows_b, sem_b).start()
        gather(c0, rows_a, sem_a).wait()
        do_chunk(c0, rows_a)
        gather(c0 + 2, rows_a, sem_a).start()
        gather(c0 + 1, rows_b, sem_b).wait()
        do_chunk(c0 + 1, rows_b)
        return carry

    lax.fori_loop(0, (NCH - 1) // 2, step, 0)

    gather(NCH - 1, rows_a, sem_a).wait()
    do_chunk(NCH - 1, rows_a)

    plsc.subcore_barrier()

    pltpu.sync_copy(acc.at[pl.ds(base, ROWS_PER_TILE)],
                    agg_out.at[cid].at[pl.ds(base, ROWS_PER_TILE)])
    pltpu.sync_copy(cnt_v, cnt_out.at[wid])


def _dense_body(agg_ref, cnt_ref, x_ref, wl_ref, bl_ref, wr_ref, o_ref, *, relu):
    agg = agg_ref[0] + agg_ref[1]
    cnt = jnp.sum(cnt_ref[...], axis=0)
    a = agg / jnp.maximum(cnt, 1.0)[:, None]
    acc = lax.dot_general(a, wl_ref[...], (((1,), (1,)), ((), ())),
                          preferred_element_type=jnp.float32)
    acc = acc + lax.dot_general(x_ref[...], wr_ref[...], (((1,), (1,)), ((), ())),
                                preferred_element_type=jnp.float32)
    acc = acc + bl_ref[...]
    if relu:
        acc = jnp.maximum(acc, 0.0)
    o_ref[...] = acc


BM = 1000


def _dense(agg, cnt, x, Wl, bl2d, Wr, relu):
    return pl.pallas_call(
        functools.partial(_dense_body, relu=relu),
        grid=(N_NODES // BM,),
        in_specs=[
            pl.BlockSpec((NC, BM, D), lambda m: (0, m, 0)),
            pl.BlockSpec((NW, BM), lambda m: (0, m)),
            pl.BlockSpec((BM, D), lambda m: (m, 0)),
            pl.BlockSpec((D, D), lambda m: (0, 0)),
            pl.BlockSpec((1, D), lambda m: (0, 0)),
            pl.BlockSpec((D, D), lambda m: (0, 0)),
        ],
        out_specs=pl.BlockSpec((BM, D), lambda m: (m, 0)),
        out_shape=jax.ShapeDtypeStruct((N_NODES, D), jnp.float32),
        compiler_params=pltpu.CompilerParams(
            dimension_semantics=("arbitrary",)),
    )(agg, cnt, x, Wl, bl2d, Wr)


def kernel(x, edge_index, W1l, b1l, W1r, W2l, b2l, W2r):
    src = edge_index[0].astype(jnp.int32).reshape(NW, NCH, CH)
    dst = edge_index[1].astype(jnp.int32).reshape(NW, NCH, CH)
    agg1, cnt = _sc_agg(x, src, dst)
    h = _dense(agg1, cnt, x, W1l, b1l.reshape(1, D), W1r, True)
    agg2, _ = _sc_agg(h, src, dst)
    out = _dense(agg2, cnt, h, W2l, b2l.reshape(1, D), W2r, False)
    return out


# speedup vs baseline: 9.5460x; 9.5460x over previous
import functools

import jax
import jax.numpy as jnp
from jax import lax
from jax.experimental import pallas as pl
from jax.experimental.pallas import tpu as pltpu
from jax.experimental.pallas import tpu_sc as plsc

N_NODES = 10000
D = 128
E = 320000

NC = 2
NS = 16
NW = NC * NS
EPW = E // NW
CH = 40
GRP = 50
NGRP = EPW // (GRP * CH)
N_PAD = 10240
ROWS_PER_TILE = N_PAD // NS

_mesh = plsc.VectorSubcoreMesh(core_axis_name="c", subcore_axis_name="s")


@functools.partial(
    pl.kernel,
    mesh=_mesh,
    out_type=(
        jax.ShapeDtypeStruct((NC, N_PAD, D), jnp.float32),
        jax.ShapeDtypeStruct((10, NW, 1, 1000), jnp.float32),
    ),
    scratch_types=[
        pltpu.VMEM((GRP, CH), jnp.int32),
        pltpu.VMEM((GRP, CH), jnp.int32),
        pltpu.VMEM((CH, D), jnp.float32),
        pltpu.VMEM((CH, D), jnp.float32),
        pltpu.VMEM((N_NODES,), jnp.float32),
        pltpu.VMEM_SHARED((N_PAD, D), jnp.float32),
        pltpu.SemaphoreType.DMA,
        pltpu.SemaphoreType.DMA,
    ],
    compiler_params=pltpu.CompilerParams(needs_layout_passes=False),
)
def _sc_agg(x_hbm, src_hbm, dst_hbm, agg_out, cnt_out,
            src_v, dst_v, rows_a, rows_b, cnt_v, acc, sem_a, sem_b):
    cid = lax.axis_index("c")
    tid = lax.axis_index("s")
    wid = cid * NS + tid

    zvec = jnp.zeros((16,), jnp.float32)

    def zero_cnt(i, carry):
        cnt_v[pl.ds(i * 16, 16)] = zvec
        return carry

    lax.fori_loop(0, N_NODES // 16, zero_cnt, 0)

    def zero_rows(r, carry):
        row = rows_a.at[r]
        for k in range(D // 16):
            row[pl.ds(k * 16, 16)] = zvec
        return carry

    lax.fori_loop(0, CH, zero_rows, 0)

    base = tid * ROWS_PER_TILE

    def zero_acc(b, carry):
        pltpu.sync_copy(rows_a, acc.at[pl.ds(base + b * CH, CH)])
        return carry

    lax.fori_loop(0, ROWS_PER_TILE // CH, zero_acc, 0)

    plsc.subcore_barrier()

    ones16 = jnp.ones((16,), jnp.float32)
    tail_mask = lax.iota(jnp.int32, 16) >= 8

    def gather(c, buf, sem):
        return pltpu.make_async_copy(x_hbm.at[src_v.at[c]], buf, sem)

    def do_chunk(c, buf):
        pltpu.sync_copy(buf, acc.at[dst_v.at[c]], add=True)
        drow = dst_v.at[c]
        plsc.addupdate_scatter(cnt_v, [drow[pl.ds(0, 16)]], ones16)
        plsc.addupdate_scatter(cnt_v, [drow[pl.ds(16, 16)]], ones16)
        plsc.addupdate_scatter(cnt_v, [drow[pl.ds(24, 16)]], ones16,
                               mask=tail_mask)

    def group(g, carry):
        pltpu.sync_copy(src_hbm.at[wid].at[g], src_v)
        pltpu.sync_copy(dst_hbm.at[wid].at[g], dst_v)
        gather(0, rows_a, sem_a).start()

        def step(t, carry2):
            c0 = 2 * t
            gather(c0 + 1, rows_b, sem_b).start()
            gather(c0, rows_a, sem_a).wait()
            do_chunk(c0, rows_a)

            @pl.when(c0 + 2 < GRP)
            def _():
                gather(c0 + 2, rows_a, sem_a).start()

            gather(c0 + 1, rows_b, sem_b).wait()
            do_chunk(c0 + 1, rows_b)
            return carry2

        lax.fori_loop(0, GRP // 2, step, 0)
        return carry

    lax.fori_loop(0, NGRP, group, 0)

    plsc.subcore_barrier()

    pltpu.sync_copy(acc.at[pl.ds(base, ROWS_PER_TILE)],
                    agg_out.at[cid].at[pl.ds(base, ROWS_PER_TILE)])

    def cnt_off(b, carry):
        pltpu.sync_copy(cnt_v.at[pl.ds(b * 1000, 1000)],
                        cnt_out.at[b].at[wid].at[0])
        return carry

    lax.fori_loop(0, N_NODES // 1000, cnt_off, 0)


def _dense_body(agg_ref, cnt_ref, x_ref, wl_ref, bl_ref, wr_ref, o_ref, *, relu):
    agg = agg_ref[0] + agg_ref[1]
    cnt = jnp.sum(cnt_ref[...], axis=(0, 1, 2))
    a = agg / jnp.maximum(cnt, 1.0)[:, None]
    acc = lax.dot_general(a, wl_ref[...], (((1,), (1,)), ((), ())),
                          preferred_element_type=jnp.float32)
    acc = acc + lax.dot_general(x_ref[...], wr_ref[...], (((1,), (1,)), ((), ())),
                                preferred_element_type=jnp.float32)
    acc = acc + bl_ref[...]
    if relu:
        acc = jnp.maximum(acc, 0.0)
    o_ref[...] = acc


BM = 1000


def _dense(agg, cnt, x, Wl, bl2d, Wr, relu):
    return pl.pallas_call(
        functools.partial(_dense_body, relu=relu),
        grid=(N_NODES // BM,),
        in_specs=[
            pl.BlockSpec((NC, BM, D), lambda m: (0, m, 0)),
            pl.BlockSpec((1, NW, 1, BM), lambda m: (m, 0, 0, 0)),
            pl.BlockSpec((BM, D), lambda m: (m, 0)),
            pl.BlockSpec((D, D), lambda m: (0, 0)),
            pl.BlockSpec((1, D), lambda m: (0, 0)),
            pl.BlockSpec((D, D), lambda m: (0, 0)),
        ],
        out_specs=pl.BlockSpec((BM, D), lambda m: (m, 0)),
        out_shape=jax.ShapeDtypeStruct((N_NODES, D), jnp.float32),
        compiler_params=pltpu.CompilerParams(
            dimension_semantics=("arbitrary",)),
    )(agg, cnt, x, Wl, bl2d, Wr)


def kernel(x, edge_index, W1l, b1l, W1r, W2l, b2l, W2r):
    src = edge_index[0].astype(jnp.int32).reshape(NW, NGRP, GRP, CH)
    dst = edge_index[1].astype(jnp.int32).reshape(NW, NGRP, GRP, CH)
    agg1, cnt = _sc_agg(x, src, dst)
    h = _dense(agg1, cnt, x, W1l, b1l.reshape(1, D), W1r, True)
    agg2, _ = _sc_agg(h, src, dst)
    out = _dense(agg2, cnt, h, W2l, b2l.reshape(1, D), W2r, False)
    return out


# speedup vs baseline: 12.0521x; 1.2625x over previous
import functools

import jax
import jax.numpy as jnp
from jax import lax
from jax.experimental import pallas as pl
from jax.experimental.pallas import tpu as pltpu
from jax.experimental.pallas import tpu_sc as plsc

N_NODES = 10000
D = 128
E = 320000

NC = 2
NS = 16
NW = NC * NS
EPW = E // NW
CH = 80
GRP = 25
NGRP = EPW // (GRP * CH)
N_PAD = 10240
ROWS_PER_TILE = N_PAD // NS

_mesh = plsc.VectorSubcoreMesh(core_axis_name="c", subcore_axis_name="s")


@functools.partial(
    pl.kernel,
    mesh=_mesh,
    out_type=(
        jax.ShapeDtypeStruct((NC, N_PAD, D), jnp.float32),
        jax.ShapeDtypeStruct((10, NW, 1, 1000), jnp.float32),
    ),
    scratch_types=[
        pltpu.VMEM((GRP, CH), jnp.int32),
        pltpu.VMEM((GRP, CH), jnp.int32),
        pltpu.VMEM((CH, D), jnp.float32),
        pltpu.VMEM((CH, D), jnp.float32),
        pltpu.VMEM((N_NODES,), jnp.float32),
        pltpu.VMEM_SHARED((N_PAD, D), jnp.float32),
        pltpu.SemaphoreType.DMA,
        pltpu.SemaphoreType.DMA,
    ],
    compiler_params=pltpu.CompilerParams(needs_layout_passes=False),
)
def _sc_agg(x_hbm, src_hbm, dst_hbm, agg_out, cnt_out,
            src_v, dst_v, rows_a, rows_b, cnt_v, acc, sem_a, sem_b):
    cid = lax.axis_index("c")
    tid = lax.axis_index("s")
    wid = cid * NS + tid

    zvec = jnp.zeros((16,), jnp.float32)

    def zero_cnt(i, carry):
        cnt_v[pl.ds(i * 16, 16)] = zvec
        return carry

    lax.fori_loop(0, N_NODES // 16, zero_cnt, 0)

    def zero_rows(r, carry):
        row = rows_a.at[r]
        for k in range(D // 16):
            row[pl.ds(k * 16, 16)] = zvec
        return carry

    lax.fori_loop(0, CH, zero_rows, 0)

    base = tid * ROWS_PER_TILE

    def zero_acc(b, carry):
        pltpu.sync_copy(rows_a, acc.at[pl.ds(base + b * CH, CH)])
        return carry

    lax.fori_loop(0, ROWS_PER_TILE // CH, zero_acc, 0)

    plsc.subcore_barrier()

    ones16 = jnp.ones((16,), jnp.float32)

    def gather(c, buf, sem):
        return pltpu.make_async_copy(x_hbm.at[src_v.at[c]], buf, sem)

    def do_chunk(c, buf):
        pltpu.sync_copy(buf, acc.at[dst_v.at[c]], add=True)
        drow = dst_v.at[c]
        for k in range(CH // 16):
            plsc.addupdate_scatter(cnt_v, [drow[pl.ds(k * 16, 16)]], ones16)

    def group(g, carry):
        pltpu.sync_copy(src_hbm.at[wid].at[g], src_v)
        pltpu.sync_copy(dst_hbm.at[wid].at[g], dst_v)
        gather(0, rows_a, sem_a).start()

        def step(t, carry2):
            c0 = 2 * t

            @pl.when(c0 + 1 < GRP)
            def _():
                gather(c0 + 1, rows_b, sem_b).start()

            gather(c0, rows_a, sem_a).wait()
            do_chunk(c0, rows_a)

            @pl.when(c0 + 2 < GRP)
            def _():
                gather(c0 + 2, rows_a, sem_a).start()

            @pl.when(c0 + 1 < GRP)
            def _():
                gather(c0 + 1, rows_b, sem_b).wait()
                do_chunk(c0 + 1, rows_b)

            return carry2

        lax.fori_loop(0, (GRP + 1) // 2, step, 0)
        return carry

    lax.fori_loop(0, NGRP, group, 0)

    plsc.subcore_barrier()

    pltpu.sync_copy(acc.at[pl.ds(base, ROWS_PER_TILE)],
                    agg_out.at[cid].at[pl.ds(base, ROWS_PER_TILE)])

    def cnt_off(b, carry):
        pltpu.sync_copy(cnt_v.at[pl.ds(b * 1000, 1000)],
                        cnt_out.at[b].at[wid].at[0])
        return carry

    lax.fori_loop(0, N_NODES // 1000, cnt_off, 0)


def _dense_body(agg_ref, cnt_ref, x_ref, wl_ref, bl_ref, wr_ref, o_ref, *, relu):
    agg = agg_ref[0] + agg_ref[1]
    cnt = jnp.sum(cnt_ref[...], axis=(0, 1, 2))
    a = agg / jnp.maximum(cnt, 1.0)[:, None]
    acc = lax.dot_general(a, wl_ref[...], (((1,), (1,)), ((), ())),
                          preferred_element_type=jnp.float32)
    acc = acc + lax.dot_general(x_ref[...], wr_ref[...], (((1,), (1,)), ((), ())),
                                preferred_element_type=jnp.float32)
    acc = acc + bl_ref[...]
    if relu:
        acc = jnp.maximum(acc, 0.0)
    o_ref[...] = acc


BM = 1000


def _dense(agg, cnt, x, Wl, bl2d, Wr, relu):
    return pl.pallas_call(
        functools.partial(_dense_body, relu=relu),
        grid=(N_NODES // BM,),
        in_specs=[
            pl.BlockSpec((NC, BM, D), lambda m: (0, m, 0)),
            pl.BlockSpec((1, NW, 1, BM), lambda m: (m, 0, 0, 0)),
            pl.BlockSpec((BM, D), lambda m: (m, 0)),
            pl.BlockSpec((D, D), lambda m: (0, 0)),
            pl.BlockSpec((1, D), lambda m: (0, 0)),
            pl.BlockSpec((D, D), lambda m: (0, 0)),
        ],
        out_specs=pl.BlockSpec((BM, D), lambda m: (m, 0)),
        out_shape=jax.ShapeDtypeStruct((N_NODES, D), jnp.float32),
        compiler_params=pltpu.CompilerParams(
            dimension_semantics=("arbitrary",)),
    )(agg, cnt, x, Wl, bl2d, Wr)


def kernel(x, edge_index, W1l, b1l, W1r, W2l, b2l, W2r):
    src = edge_index[0].astype(jnp.int32).reshape(NW, NGRP, GRP, CH)
    dst = edge_index[1].astype(jnp.int32).reshape(NW, NGRP, GRP, CH)
    agg1, cnt = _sc_agg(x, src, dst)
    h = _dense(agg1, cnt, x, W1l, b1l.reshape(1, D), W1r, True)
    agg2, _ = _sc_agg(h, src, dst)
    out = _dense(agg2, cnt, h, W2l, b2l.reshape(1, D), W2r, False)
    return out


# speedup vs baseline: 13.4204x; 1.1135x over previous
import functools

import jax
import jax.numpy as jnp
from jax import lax
from jax.experimental import pallas as pl
from jax.experimental.pallas import tpu as pltpu
from jax.experimental.pallas import tpu_sc as plsc

N_NODES = 10000
D = 128
E = 320000

NC = 2
NS = 16
NW = NC * NS
EPW = E // NW
CH = 40
GRP = 50
NGRP = EPW // (GRP * CH)
NBUF = 4
N_PAD = 10240
ROWS_PER_TILE = N_PAD // NS

_mesh = plsc.VectorSubcoreMesh(core_axis_name="c", subcore_axis_name="s")


@functools.partial(
    pl.kernel,
    mesh=_mesh,
    out_type=(
        jax.ShapeDtypeStruct((NC, N_PAD, D), jnp.float32),
        jax.ShapeDtypeStruct((10, NW, 1, 1000), jnp.float32),
    ),
    scratch_types=[
        pltpu.VMEM((GRP, CH), jnp.int32),
        pltpu.VMEM((GRP, CH), jnp.int32),
        [pltpu.VMEM((CH, D), jnp.float32)] * NBUF,
        pltpu.VMEM((N_NODES,), jnp.float32),
        pltpu.VMEM_SHARED((N_PAD, D), jnp.float32),
        [pltpu.SemaphoreType.DMA] * NBUF,
        [pltpu.SemaphoreType.DMA] * NBUF,
    ],
    compiler_params=pltpu.CompilerParams(needs_layout_passes=False),
)
def _sc_agg(x_hbm, src_hbm, dst_hbm, agg_out, cnt_out,
            src_v, dst_v, rows, cnt_v, acc, gsems, ssems):
    cid = lax.axis_index("c")
    tid = lax.axis_index("s")
    wid = cid * NS + tid

    zvec = jnp.zeros((16,), jnp.float32)

    def zero_cnt(i, carry):
        cnt_v[pl.ds(i * 16, 16)] = zvec
        return carry

    lax.fori_loop(0, N_NODES // 16, zero_cnt, 0)

    def zero_rows(r, carry):
        row = rows[0].at[r]
        for k in range(D // 16):
            row[pl.ds(k * 16, 16)] = zvec
        return carry

    lax.fori_loop(0, CH, zero_rows, 0)

    base = tid * ROWS_PER_TILE

    def zero_acc(b, carry):
        pltpu.sync_copy(rows[0], acc.at[pl.ds(base + b * CH, CH)])
        return carry

    lax.fori_loop(0, ROWS_PER_TILE // CH, zero_acc, 0)

    plsc.subcore_barrier()

    ones16 = jnp.ones((16,), jnp.float32)
    tail_mask = lax.iota(jnp.int32, 16) >= 8

    def gather(c, b):
        return pltpu.make_async_copy(x_hbm.at[src_v.at[c]], rows[b], gsems[b])

    def scat(c, b):
        return pltpu.make_async_copy(rows[b], acc.at[dst_v.at[c]], ssems[b])

    def counts(c):
        drow = dst_v.at[c]
        plsc.addupdate_scatter(cnt_v, [drow[pl.ds(0, 16)]], ones16)
        plsc.addupdate_scatter(cnt_v, [drow[pl.ds(16, 16)]], ones16)
        plsc.addupdate_scatter(cnt_v, [drow[pl.ds(24, 16)]], ones16,
                               mask=tail_mask)

    def group(g, carry):
        pltpu.sync_copy(src_hbm.at[wid].at[g], src_v)
        pltpu.sync_copy(dst_hbm.at[wid].at[g], dst_v)
        for b in range(NBUF):
            gather(b, b).start()

        def step(t, carry2):
            for b in range(NBUF):
                c = NBUF * t + b

                @pl.when(c < GRP)
                def _():
                    gather(c, b).wait()
                    scat(c, b).start(add=True)
                    counts(c)

                @pl.when(c + NBUF < GRP)
                def _():
                    scat(c, b).wait()
                    gather(c + NBUF, b).start()

            return carry2

        lax.fori_loop(0, (GRP + NBUF - 1) // NBUF, step, 0)

        for b in range(NBUF):
            c_last = [c for c in range(GRP) if c % NBUF == b][-1]
            scat(c_last, b).wait()
        return carry

    lax.fori_loop(0, NGRP, group, 0)

    plsc.subcore_barrier()

    pltpu.sync_copy(acc.at[pl.ds(base, ROWS_PER_TILE)],
                    agg_out.at[cid].at[pl.ds(base, ROWS_PER_TILE)])

    def cnt_off(b, carry):
        pltpu.sync_copy(cnt_v.at[pl.ds(b * 1000, 1000)],
                        cnt_out.at[b].at[wid].at[0])
        return carry

    lax.fori_loop(0, N_NODES // 1000, cnt_off, 0)


def _dense_body(agg_ref, cnt_ref, x_ref, wl_ref, bl_ref, wr_ref, o_ref, *, relu):
    agg = agg_ref[0] + agg_ref[1]
    cnt = jnp.sum(cnt_ref[...], axis=(0, 1, 2))
    a = agg / jnp.maximum(cnt, 1.0)[:, None]
    acc = lax.dot_general(a, wl_ref[...], (((1,), (1,)), ((), ())),
                          preferred_element_type=jnp.float32)
    acc = acc + lax.dot_general(x_ref[...], wr_ref[...], (((1,), (1,)), ((), ())),
                                preferred_element_type=jnp.float32)
    acc = acc + bl_ref[...]
    if relu:
        acc = jnp.maximum(acc, 0.0)
    o_ref[...] = acc


BM = 1000


def _dense(agg, cnt, x, Wl, bl2d, Wr, relu):
    return pl.pallas_call(
        functools.partial(_dense_body, relu=relu),
        grid=(N_NODES // BM,),
        in_specs=[
            pl.BlockSpec((NC, BM, D), lambda m: (0, m, 0)),
            pl.BlockSpec((1, NW, 1, BM), lambda m: (m, 0, 0, 0)),
            pl.BlockSpec((BM, D), lambda m: (m, 0)),
            pl.BlockSpec((D, D), lambda m: (0, 0)),
            pl.BlockSpec((1, D), lambda m: (0, 0)),
            pl.BlockSpec((D, D), lambda m: (0, 0)),
        ],
        out_specs=pl.BlockSpec((BM, D), lambda m: (m, 0)),
        out_shape=jax.ShapeDtypeStruct((N_NODES, D), jnp.float32),
        compiler_params=pltpu.CompilerParams(
            dimension_semantics=("arbitrary",)),
    )(agg, cnt, x, Wl, bl2d, Wr)


def kernel(x, edge_index, W1l, b1l, W1r, W2l, b2l, W2r):
    src = edge_index[0].astype(jnp.int32).reshape(NW, NGRP, GRP, CH)
    dst = edge_index[1].astype(jnp.int32).reshape(NW, NGRP, GRP, CH)
    agg1, cnt = _sc_agg(x, src, dst)
    h = _dense(agg1, cnt, x, W1l, b1l.reshape(1, D), W1r, True)
    agg2, _ = _sc_agg(h, src, dst)
    out = _dense(agg2, cnt, h, W2l, b2l.reshape(1, D), W2r, False)
    return out
